# baseline (speedup 1.0000x reference)
import functools

import jax
import jax.numpy as jnp
from jax import lax
from jax.experimental import pallas as pl
from jax.experimental.pallas import tpu as pltpu
from jax.experimental.pallas import tpu_sc as plsc

ROWS = 128
COLS = 32768
NW = 32
RT = 8
NRT = ROWS // RT
CT = COLS // 128
SC_CT = CT // 4
SC_COLS = SC_CT * 128
RPW = 4
CHT = 32
NCHUNK = SC_CT // CHT
NEG_INF = float("-inf")
I32_MAX = 2147483647

TC_BLK = 1024
TC_NBLK = (COLS - SC_COLS) // TC_BLK


def _combine(av, ai, bv, bi):
    take_b = (bv > av) | ((bv == av) & (bi < ai))
    return jnp.where(take_b, bv, av), jnp.where(take_b, bi, ai)


def _topk1_sc(x4):
    mesh = plsc.VectorSubcoreMesh(core_axis_name="c", subcore_axis_name="s")

    @functools.partial(
        pl.kernel,
        mesh=mesh,
        out_type=(
            jax.ShapeDtypeStruct((NW * 16,), jnp.float32),
            jax.ShapeDtypeStruct((NW * 16,), jnp.int32),
        ),
        scratch_types=[
            pltpu.VMEM((CHT, RPW, 128), jnp.float32),
            pltpu.VMEM((CHT, RPW, 128), jnp.float32),
            pltpu.VMEM((RPW, CHT, 1, 128), jnp.float32),
            pltpu.VMEM((16,), jnp.float32),
            pltpu.VMEM((16,), jnp.int32),
            pltpu.SemaphoreType.DMA,
            pltpu.SemaphoreType.DMA,
            pltpu.SemaphoreType.DMA,
        ],
        compiler_params=pltpu.CompilerParams(skip_device_barrier=True),
    )
    def k(x_hbm, outv_hbm, outi_hbm, buf0, buf1, pbuf, stg_v, stg_i,
          sem0, sem1, sem2):
        cid = lax.axis_index("c")
        sid = lax.axis_index("s")
        wid = cid * 16 + sid
        rt = wid // 2
        r0 = (wid % 2) * RPW
        lane = lax.iota(jnp.int32, 16)

        bufs = (buf0, buf1)
        sems = (sem0, sem1)

        def src(t):
            return x_hbm.at[rt, pl.ds(t * CHT, CHT), pl.ds(r0, RPW)]

        handles = [None] * NCHUNK
        handles[0] = pltpu.async_copy(src(0), bufs[0], sems[0])

        gval = [jnp.full((16,), NEG_INF, jnp.float32) for _ in range(8)]
        gch = [jnp.zeros((16,), jnp.int32) for _ in range(8)]

        for t in range(NCHUNK):
            if t + 1 < NCHUNK:
                s = (t + 1) % 2
                handles[t + 1] = pltpu.async_copy(src(t + 1), bufs[s], sems[s])
            handles[t].wait()
            buf = bufs[t % 2]

            def body(i, carry, buf=buf):
                acc = list(carry)
                for u in range(2):
                    for j in range(8):
                        for r in range(RPW):
                            q = r * 2 + (j % 2)
                            acc[q] = jnp.maximum(
                                acc[q], buf[i * 2 + u, r, pl.ds(j * 16, 16)])
                return tuple(acc)

            init = tuple(jnp.full((16,), NEG_INF, jnp.float32)
                         for _ in range(8))
            cacc = lax.fori_loop(0, CHT // 2, body, init)
            for q in range(8):
                m = cacc[q] > gval[q]
                gval[q] = jnp.where(m, cacc[q], gval[q])
                gch[q] = jnp.where(m, t, gch[q])

        mvecs, cvec_all = [], jnp.zeros((16,), jnp.int32)
        for r in range(RPW):
            v, c = _combine(gval[r * 2], gch[r * 2],
                            gval[r * 2 + 1], gch[r * 2 + 1])
            for dist in (1, 2, 4, 8):
                perm = lane ^ dist
                pv = v.at[perm].get(mode="promise_in_bounds")
                pc = c.at[perm].get(mode="promise_in_bounds")
                v, c = _combine(v, c, pv, pc)
            mvecs.append(v)
            cvec_all = jnp.where(lane == r, c, cvec_all)

        chunk_ids = [cvec_all[r] for r in range(RPW)]
        ph = []
        for r in range(RPW):
            cs = chunk_ids[r]
            ph.append(pltpu.async_copy(
                x_hbm.at[rt, pl.ds(cs * CHT, CHT), pl.ds(r0 + r, 1)],
                pbuf.at[r], sem2))

        res_v = jnp.zeros((16,), jnp.float32)
        res_i = jnp.zeros((16,), jnp.int32)
        for r in range(RPW):
            ph[r].wait()
            cs = chunk_ids[r]
            cur0 = lane + cs * (CHT * 128)

            def body2(i, carry, r=r, mv=mvecs[r]):
                cur, imin = carry
                for j in range(8):
                    v = pbuf[r, i, 0, pl.ds(j * 16, 16)]
                    eq = v == mv
                    cand = jnp.where(eq, cur + j * 16, I32_MAX)
                    imin = jnp.minimum(imin, cand)
                return (cur + 128, imin)

            _, imin = lax.fori_loop(
                0, CHT, body2, (cur0, jnp.full((16,), I32_MAX, jnp.int32)))
            for dist in (1, 2, 4, 8):
                pm = imin.at[lane ^ dist].get(mode="promise_in_bounds")
                imin = jnp.minimum(imin, pm)
            res_v = jnp.where(lane == r, mvecs[r], res_v)
            res_i = jnp.where(lane == r, imin, res_i)

        stg_v[...] = res_v
        stg_i[...] = res_i
        pltpu.sync_copy(stg_v, outv_hbm.at[pl.ds(wid * 16, 16)])
        pltpu.sync_copy(stg_i, outi_hbm.at[pl.ds(wid * 16, 16)])

    return k(x4)


def _tc_half_body(x_ref, outv_ref, outi_ref, acc_v, acc_t):
    t = pl.program_id(0)
    blk = x_ref[...]

    @pl.when(t == 0)
    def _():
        acc_v[...] = blk
        acc_t[...] = jnp.zeros((ROWS, TC_BLK), jnp.int32)

    @pl.when(t > 0)
    def _():
        m = blk > acc_v[...]
        acc_v[...] = jnp.where(m, blk, acc_v[...])
        acc_t[...] = jnp.where(m, t, acc_t[...])

    @pl.when(t == TC_NBLK - 1)
    def _():
        av = acc_v[...]
        rowmax = jnp.max(av, axis=1, keepdims=True)
        pos = jax.lax.broadcasted_iota(jnp.int32, (ROWS, TC_BLK), 1)
        cols = acc_t[...] * TC_BLK + pos + SC_COLS
        cand = jnp.where(av == rowmax, cols, I32_MAX)
        outv_ref[...] = rowmax
        outi_ref[...] = jnp.min(cand, axis=1, keepdims=True)


def _topk1_tc_half(x):
    return pl.pallas_call(
        _tc_half_body,
        grid=(TC_NBLK,),
        in_specs=[pl.BlockSpec((ROWS, TC_BLK),
                               lambda t: (0, (SC_COLS // TC_BLK) + t))],
        out_specs=(pl.BlockSpec((ROWS, 1), lambda t: (0, 0)),
                   pl.BlockSpec((ROWS, 1), lambda t: (0, 0))),
        out_shape=(jax.ShapeDtypeStruct((ROWS, 1), jnp.float32),
                   jax.ShapeDtypeStruct((ROWS, 1), jnp.int32)),
        scratch_shapes=[pltpu.VMEM((ROWS, TC_BLK), jnp.float32),
                        pltpu.VMEM((ROWS, TC_BLK), jnp.int32)],
    )(x)


def _merge_body(av_ref, ai_ref, bv_ref, bi_ref, outv_ref, outi_ref):
    mv, mi = _combine(av_ref[...], ai_ref[...], bv_ref[...], bi_ref[...])
    outv_ref[...] = mv
    outi_ref[...] = mi


def _merge(av, ai, bv, bi):
    return pl.pallas_call(
        _merge_body,
        out_shape=(jax.ShapeDtypeStruct((ROWS, 1), jnp.float32),
                   jax.ShapeDtypeStruct((ROWS, 1), jnp.int32)),
    )(av, ai, bv, bi)


def kernel(x):
    x4 = x.reshape(NRT, RT, CT, 128).swapaxes(1, 2)
    scv, sci = _topk1_sc(x4)
    tcv, tci = _topk1_tc_half(x)
    av = scv.reshape(NW, 16)[:, :RPW].reshape(ROWS, 1)
    ai = sci.reshape(NW, 16)[:, :RPW].reshape(ROWS, 1)
    vals, idxs = _merge(av, ai, tcv, tci)
    return vals, idxs


# speedup vs baseline: 1.1418x; 1.1418x over previous
import functools

import jax
import jax.numpy as jnp
from jax import lax
from jax.experimental import pallas as pl
from jax.experimental.pallas import tpu as pltpu
from jax.experimental.pallas import tpu_sc as plsc

ROWS = 128
COLS = 32768
NW = 32
RT = 8
NRT = ROWS // RT
CT = COLS // 128
SC_CT = CT // 2
SC_COLS = SC_CT * 128
RPW = 4
CHT = 32
NCHUNK = SC_CT // CHT
NEG_INF = float("-inf")
I32_MAX = 2147483647

TC_BLK = 1024
TC_NBLK = (COLS - SC_COLS) // TC_BLK


def _combine(av, ai, bv, bi):
    take_b = (bv > av) | ((bv == av) & (bi < ai))
    return jnp.where(take_b, bv, av), jnp.where(take_b, bi, ai)


def _topk1_sc(x4):
    mesh = plsc.VectorSubcoreMesh(core_axis_name="c", subcore_axis_name="s")

    @functools.partial(
        pl.kernel,
        mesh=mesh,
        out_type=(
            jax.ShapeDtypeStruct((NW * 16,), jnp.float32),
            jax.ShapeDtypeStruct((NW * 16,), jnp.int32),
        ),
        scratch_types=[
            pltpu.VMEM((CHT, RPW, 128), jnp.float32),
            pltpu.VMEM((CHT, RPW, 128), jnp.float32),
            pltpu.VMEM((RPW, CHT, 1, 128), jnp.float32),
            pltpu.VMEM((16,), jnp.float32),
            pltpu.VMEM((16,), jnp.int32),
            pltpu.SemaphoreType.DMA,
            pltpu.SemaphoreType.DMA,
            pltpu.SemaphoreType.DMA,
        ],
        compiler_params=pltpu.CompilerParams(skip_device_barrier=True),
    )
    def k(x_hbm, outv_hbm, outi_hbm, buf0, buf1, pbuf, stg_v, stg_i,
          sem0, sem1, sem2):
        cid = lax.axis_index("c")
        sid = lax.axis_index("s")
        wid = cid * 16 + sid
        rt = wid // 2
        r0 = (wid % 2) * RPW
        lane = lax.iota(jnp.int32, 16)

        bufs = (buf0, buf1)
        sems = (sem0, sem1)

        def src(t):
            return x_hbm.at[rt, pl.ds(t * CHT, CHT), pl.ds(r0, RPW)]

        handles = [None] * NCHUNK
        handles[0] = pltpu.async_copy(src(0), bufs[0], sems[0])

        gval = [jnp.full((16,), NEG_INF, jnp.float32) for _ in range(8)]
        gch = [jnp.zeros((16,), jnp.int32) for _ in range(8)]

        for t in range(NCHUNK):
            if t + 1 < NCHUNK:
                s = (t + 1) % 2
                handles[t + 1] = pltpu.async_copy(src(t + 1), bufs[s], sems[s])
            handles[t].wait()
            buf = bufs[t % 2]

            def body(i, carry, buf=buf):
                acc = list(carry)
                for u in range(2):
                    for j in range(8):
                        for r in range(RPW):
                            q = r * 2 + (j % 2)
                            acc[q] = jnp.maximum(
                                acc[q], buf[i * 2 + u, r, pl.ds(j * 16, 16)])
                return tuple(acc)

            init = tuple(jnp.full((16,), NEG_INF, jnp.float32)
                         for _ in range(8))
            cacc = lax.fori_loop(0, CHT // 2, body, init)
            for q in range(8):
                m = cacc[q] > gval[q]
                gval[q] = jnp.where(m, cacc[q], gval[q])
                gch[q] = jnp.where(m, t, gch[q])

        mvecs, cvec_all = [], jnp.zeros((16,), jnp.int32)
        for r in range(RPW):
            v, c = _combine(gval[r * 2], gch[r * 2],
                            gval[r * 2 + 1], gch[r * 2 + 1])
            for dist in (1, 2, 4, 8):
                perm = lane ^ dist
                pv = v.at[perm].get(mode="promise_in_bounds")
                pc = c.at[perm].get(mode="promise_in_bounds")
                v, c = _combine(v, c, pv, pc)
            mvecs.append(v)
            cvec_all = jnp.where(lane == r, c, cvec_all)

        chunk_ids = [cvec_all[r] for r in range(RPW)]
        ph = []
        for r in range(RPW):
            cs = chunk_ids[r]
            ph.append(pltpu.async_copy(
                x_hbm.at[rt, pl.ds(cs * CHT, CHT), pl.ds(r0 + r, 1)],
                pbuf.at[r], sem2))

        res_v = jnp.zeros((16,), jnp.float32)
        res_i = jnp.zeros((16,), jnp.int32)
        for r in range(RPW):
            ph[r].wait()
            cs = chunk_ids[r]
            cur0 = lane + cs * (CHT * 128)

            def body2(i, carry, r=r, mv=mvecs[r]):
                cur, imin = carry
                for j in range(8):
                    v = pbuf[r, i, 0, pl.ds(j * 16, 16)]
                    eq = v == mv
                    cand = jnp.where(eq, cur + j * 16, I32_MAX)
                    imin = jnp.minimum(imin, cand)
                return (cur + 128, imin)

            _, imin = lax.fori_loop(
                0, CHT, body2, (cur0, jnp.full((16,), I32_MAX, jnp.int32)))
            for dist in (1, 2, 4, 8):
                pm = imin.at[lane ^ dist].get(mode="promise_in_bounds")
                imin = jnp.minimum(imin, pm)
            res_v = jnp.where(lane == r, mvecs[r], res_v)
            res_i = jnp.where(lane == r, imin, res_i)

        stg_v[...] = res_v
        stg_i[...] = res_i
        pltpu.sync_copy(stg_v, outv_hbm.at[pl.ds(wid * 16, 16)])
        pltpu.sync_copy(stg_i, outi_hbm.at[pl.ds(wid * 16, 16)])

    return k(x4)


def _tc_half_body(x_ref, outv_ref, outi_ref, acc_v, acc_t):
    t = pl.program_id(0)
    blk = x_ref[...]

    @pl.when(t == 0)
    def _():
        acc_v[...] = blk
        acc_t[...] = jnp.zeros((ROWS, TC_BLK), jnp.int32)

    @pl.when(t > 0)
    def _():
        m = blk > acc_v[...]
        acc_v[...] = jnp.where(m, blk, acc_v[...])
        acc_t[...] = jnp.where(m, t, acc_t[...])

    @pl.when(t == TC_NBLK - 1)
    def _():
        av = acc_v[...]
        rowmax = jnp.max(av, axis=1, keepdims=True)
        pos = jax.lax.broadcasted_iota(jnp.int32, (ROWS, TC_BLK), 1)
        cols = acc_t[...] * TC_BLK + pos + SC_COLS
        cand = jnp.where(av == rowmax, cols, I32_MAX)
        rowidx = jnp.min(cand, axis=1, keepdims=True)
        outv_ref[...] = rowmax.reshape(1, ROWS)
        outi_ref[...] = rowidx.reshape(1, ROWS)


def _topk1_tc_half(x):
    return pl.pallas_call(
        _tc_half_body,
        grid=(TC_NBLK,),
        in_specs=[pl.BlockSpec((ROWS, TC_BLK),
                               lambda t: (0, (SC_COLS // TC_BLK) + t))],
        out_specs=(pl.BlockSpec((1, ROWS), lambda t: (0, 0)),
                   pl.BlockSpec((1, ROWS), lambda t: (0, 0))),
        out_shape=(jax.ShapeDtypeStruct((1, ROWS), jnp.float32),
                   jax.ShapeDtypeStruct((1, ROWS), jnp.int32)),
        scratch_shapes=[pltpu.VMEM((ROWS, TC_BLK), jnp.float32),
                        pltpu.VMEM((ROWS, TC_BLK), jnp.int32)],
    )(x)


def _merge_body(av_ref, ai_ref, bv_ref, bi_ref, outv_ref, outi_ref):
    mv, mi = _combine(av_ref[...], ai_ref[...], bv_ref[...], bi_ref[...])
    outv_ref[...] = mv
    outi_ref[...] = mi


def _merge(av, ai, bv, bi):
    return pl.pallas_call(
        _merge_body,
        out_shape=(jax.ShapeDtypeStruct((1, ROWS), jnp.float32),
                   jax.ShapeDtypeStruct((1, ROWS), jnp.int32)),
    )(av, ai, bv, bi)


_SC_PERM = jnp.asarray(
    [(r // RPW) * 16 + r % RPW for r in range(ROWS)], dtype=jnp.int32)


def kernel(x):
    x4 = x.reshape(NRT, RT, CT, 128).swapaxes(1, 2)
    scv, sci = _topk1_sc(x4)
    tcv, tci = _topk1_tc_half(x)
    av = scv[_SC_PERM].reshape(1, ROWS)
    ai = sci[_SC_PERM].reshape(1, ROWS)
    vals, idxs = _merge(av, ai, tcv, tci)
    return vals.reshape(ROWS, 1), idxs.reshape(ROWS, 1)
